# baseline (speedup 1.0000x reference)
import functools

import jax
import jax.numpy as jnp
from jax.experimental import pallas as pl
from jax.experimental.pallas import tpu as pltpu

NUM_HEADS = 12
TOP_K = 2
NUM_EXPERTS = 8
D_MODEL = 768
D_FF = 1024
HEAD_DIM = D_MODEL // NUM_HEADS
SEQ = 2048

BQ = 2048
BK = 1024
BT = 256


def _attn_body(q_ref, k_ref, o_ref, m_ref, d_ref):
    kt = pl.program_id(2)

    @pl.when(kt == 0)
    def _():
        o_ref[0] = jnp.zeros((BQ, HEAD_DIM), jnp.float32)
        m_ref[...] = jnp.full((BQ, 1), -jnp.inf, jnp.float32)
        d_ref[...] = jnp.zeros((BQ, 1), jnp.float32)

    q = q_ref[0]
    k = k_ref[0]
    s = jax.lax.dot_general(q, k, (((1,), (1,)), ((), ())),
                            preferred_element_type=jnp.float32) * 0.125
    m_tile = jnp.max(s, axis=1, keepdims=True)
    m_old = m_ref[...]
    m_new = jnp.maximum(m_old, m_tile)
    delta = jnp.where(m_old == m_new, 0.0, m_old - m_new)
    p = jnp.exp(s - m_new)
    sum_tile = jnp.sum(p, axis=1, keepdims=True)
    scale = jnp.exp(delta)
    d_old = d_ref[...]
    coef = scale * d_old
    d_new = coef + sum_tile
    o_rescaled = coef * o_ref[0]
    o_unnorm = o_rescaled + jax.lax.dot_general(
        p, k, (((1,), (0,)), ((), ())), preferred_element_type=jnp.float32)
    o_ref[0] = o_unnorm * (1.0 / d_new)
    m_ref[...] = m_new
    d_ref[...] = d_new


def _attention(xh):
    grid = (NUM_HEADS, SEQ // BQ, SEQ // BK)
    return pl.pallas_call(
        _attn_body,
        grid=grid,
        in_specs=[
            pl.BlockSpec((1, BQ, HEAD_DIM), lambda h, q, kt: (h, q, 0)),
            pl.BlockSpec((1, BK, HEAD_DIM), lambda h, q, kt: (h, kt, 0)),
        ],
        out_specs=pl.BlockSpec((1, BQ, HEAD_DIM), lambda h, q, kt: (h, q, 0)),
        out_shape=jax.ShapeDtypeStruct((NUM_HEADS, SEQ, HEAD_DIM),
                                       jnp.float32),
        scratch_shapes=[pltpu.VMEM((BQ, 1), jnp.float32),
                        pltpu.VMEM((BQ, 1), jnp.float32)],
        compiler_params=pltpu.CompilerParams(
            dimension_semantics=("parallel", "parallel", "arbitrary")),
    )(xh, xh)


def _gate_body(xf_ref, wg_ref, fw_ref):
    xf = xf_ref[...]
    wg = wg_ref[...]
    logits = jax.lax.dot_general(xf, wg, (((1,), (1,)), ((), ())),
                                 preferred_element_type=jnp.float32)
    idx = jax.lax.broadcasted_iota(jnp.int32, logits.shape, 1)
    m1 = jnp.max(logits, axis=1, keepdims=True)
    i1 = jnp.min(jnp.where(logits == m1, idx, NUM_EXPERTS), axis=1,
                 keepdims=True)
    masked = jnp.where(idx == i1, -jnp.inf, logits)
    m2 = jnp.max(masked, axis=1, keepdims=True)
    i2 = jnp.min(jnp.where(masked == m2, idx, NUM_EXPERTS), axis=1,
                 keepdims=True)
    w1 = 1.0 / (1.0 + jnp.exp(m2 - m1))
    w2 = 1.0 - w1
    fw_ref[...] = (jnp.where(idx == i1, w1, 0.0)
                   + jnp.where(idx == i2, w2, 0.0))


def _gating(xf, Wg):
    return pl.pallas_call(
        _gate_body,
        out_shape=jax.ShapeDtypeStruct((SEQ, NUM_EXPERTS), jnp.float32),
    )(xf, Wg)


def _moe_body(xf_ref, fw_ref, w1_ref, b1_ref, w2_ref, b2_ref, o_ref):
    xf = xf_ref[...].astype(jnp.bfloat16)
    fw = fw_ref[...]
    lane = jax.lax.broadcasted_iota(jnp.int32, fw.shape, 1)
    acc = jnp.zeros((BT, D_MODEL), jnp.float32)
    for e in range(NUM_EXPERTS):
        h = jax.lax.dot_general(xf, w1_ref[e], (((1,), (1,)), ((), ())),
                                preferred_element_type=jnp.float32)
        h = h + b1_ref[e][None, :]
        h = (h * 0.5 * (1.0 + jax.lax.erf(h * 0.7071067811865476))
             ).astype(jnp.bfloat16)
        eo = jax.lax.dot_general(h, w2_ref[e], (((1,), (1,)), ((), ())),
                                 preferred_element_type=jnp.float32)
        eo = eo + b2_ref[e][None, :]
        wcol = jnp.sum(jnp.where(lane == e, fw, 0.0), axis=1, keepdims=True)
        acc = acc + eo * wcol
    o_ref[...] = acc


def _moe_dense(xf, fw, W1, b1, W2, b2):
    grid = (SEQ // BT,)
    return pl.pallas_call(
        _moe_body,
        grid=grid,
        in_specs=[
            pl.BlockSpec((BT, D_MODEL), lambda t: (t, 0)),
            pl.BlockSpec((BT, NUM_EXPERTS), lambda t: (t, 0)),
            pl.BlockSpec((NUM_EXPERTS, D_FF, D_MODEL), lambda t: (0, 0, 0)),
            pl.BlockSpec((NUM_EXPERTS, D_FF), lambda t: (0, 0)),
            pl.BlockSpec((NUM_EXPERTS, D_MODEL, D_FF), lambda t: (0, 0, 0)),
            pl.BlockSpec((NUM_EXPERTS, D_MODEL), lambda t: (0, 0)),
        ],
        out_specs=pl.BlockSpec((BT, D_MODEL), lambda t: (t, 0)),
        out_shape=jax.ShapeDtypeStruct((SEQ, D_MODEL), jnp.float32),
        compiler_params=pltpu.CompilerParams(
            dimension_semantics=("arbitrary",)),
    )(xf, fw, W1, b1, W2, b2)


@jax.jit
def kernel(x, Wg, W1, b1, W2, b2):
    xh = x.reshape(SEQ, NUM_HEADS, HEAD_DIM).transpose(1, 0, 2)
    attn = _attention(xh).transpose(1, 0, 2).reshape(SEQ, D_MODEL)
    fw = _gating(attn, Wg)
    out = _moe_dense(attn, fw,
                     W1.astype(jnp.bfloat16), b1,
                     W2.astype(jnp.bfloat16), b2)
    return out.reshape(1, SEQ, D_MODEL)


# speedup vs baseline: 1.0515x; 1.0515x over previous
import functools

import jax
import jax.numpy as jnp
from jax.experimental import pallas as pl
from jax.experimental.pallas import tpu as pltpu

NUM_HEADS = 12
TOP_K = 2
NUM_EXPERTS = 8
D_MODEL = 768
D_FF = 1024
HEAD_DIM = D_MODEL // NUM_HEADS
SEQ = 2048

BQ = 2048
BK = 1024
BT = 256


def _attn_body(q_ref, k_ref, o_ref, m_ref, d_ref):
    kt = pl.program_id(2)

    @pl.when(kt == 0)
    def _():
        o_ref[0] = jnp.zeros((BQ, HEAD_DIM), jnp.float32)
        m_ref[...] = jnp.full((BQ, 1), -jnp.inf, jnp.float32)
        d_ref[...] = jnp.zeros((BQ, 1), jnp.float32)

    q = q_ref[0]
    k = k_ref[0]
    s = jax.lax.dot_general(q, k, (((1,), (1,)), ((), ())),
                            preferred_element_type=jnp.float32) * 0.125
    m_tile = jnp.max(s, axis=1, keepdims=True)
    m_old = m_ref[...]
    m_new = jnp.maximum(m_old, m_tile)
    delta = jnp.where(m_old == m_new, 0.0, m_old - m_new)
    p = jnp.exp(s - m_new)
    sum_tile = jnp.sum(p, axis=1, keepdims=True)
    scale = jnp.exp(delta)
    d_old = d_ref[...]
    coef = scale * d_old
    d_new = coef + sum_tile
    o_rescaled = coef * o_ref[0]
    o_unnorm = o_rescaled + jax.lax.dot_general(
        p, k, (((1,), (0,)), ((), ())), preferred_element_type=jnp.float32)
    o_ref[0] = o_unnorm * (1.0 / d_new)
    m_ref[...] = m_new
    d_ref[...] = d_new


def _attention(xh):
    grid = (NUM_HEADS, SEQ // BQ, SEQ // BK)
    return pl.pallas_call(
        _attn_body,
        grid=grid,
        in_specs=[
            pl.BlockSpec((1, BQ, HEAD_DIM), lambda h, q, kt: (h, q, 0)),
            pl.BlockSpec((1, BK, HEAD_DIM), lambda h, q, kt: (h, kt, 0)),
        ],
        out_specs=pl.BlockSpec((1, BQ, HEAD_DIM), lambda h, q, kt: (h, q, 0)),
        out_shape=jax.ShapeDtypeStruct((NUM_HEADS, SEQ, HEAD_DIM),
                                       jnp.float32),
        scratch_shapes=[pltpu.VMEM((BQ, 1), jnp.float32),
                        pltpu.VMEM((BQ, 1), jnp.float32)],
        compiler_params=pltpu.CompilerParams(
            dimension_semantics=("parallel", "parallel", "arbitrary")),
    )(xh, xh)


def _gate_body(xf_ref, wg_ref, fw_ref):
    xf = xf_ref[...]
    wg = wg_ref[...]
    logits = jax.lax.dot_general(xf, wg, (((1,), (1,)), ((), ())),
                                 preferred_element_type=jnp.float32)
    idx = jax.lax.broadcasted_iota(jnp.int32, logits.shape, 1)
    m1 = jnp.max(logits, axis=1, keepdims=True)
    i1 = jnp.min(jnp.where(logits == m1, idx, NUM_EXPERTS), axis=1,
                 keepdims=True)
    masked = jnp.where(idx == i1, -jnp.inf, logits)
    m2 = jnp.max(masked, axis=1, keepdims=True)
    i2 = jnp.min(jnp.where(masked == m2, idx, NUM_EXPERTS), axis=1,
                 keepdims=True)
    w1 = 1.0 / (1.0 + jnp.exp(m2 - m1))
    w2 = 1.0 - w1
    fw_ref[...] = (jnp.where(idx == i1, w1, 0.0)
                   + jnp.where(idx == i2, w2, 0.0))


def _gating(xf, Wg):
    return pl.pallas_call(
        _gate_body,
        out_shape=jax.ShapeDtypeStruct((SEQ, NUM_EXPERTS), jnp.float32),
    )(xf, Wg)


def _moe_body(xf_ref, fw_ref, w1_ref, b1_ref, w2_ref, b2_ref, o_ref):
    xf = xf_ref[...].astype(jnp.bfloat16)
    fw = fw_ref[...]
    lane = jax.lax.broadcasted_iota(jnp.int32, fw.shape, 1)
    acc = jnp.zeros((BT, D_MODEL), jnp.float32)
    for e in range(NUM_EXPERTS):
        w1 = w1_ref[e].astype(jnp.bfloat16)
        h = jax.lax.dot_general(xf, w1, (((1,), (1,)), ((), ())),
                                preferred_element_type=jnp.float32)
        h = h + b1_ref[e][None, :]
        h = (h * 0.5 * (1.0 + jax.lax.erf(h * 0.7071067811865476))
             ).astype(jnp.bfloat16)
        w2 = w2_ref[e].astype(jnp.bfloat16)
        eo = jax.lax.dot_general(h, w2, (((1,), (1,)), ((), ())),
                                 preferred_element_type=jnp.float32)
        eo = eo + b2_ref[e][None, :]
        wcol = jnp.sum(jnp.where(lane == e, fw, 0.0), axis=1, keepdims=True)
        acc = acc + eo * wcol
    o_ref[...] = acc


def _moe_dense(xf, fw, W1, b1, W2, b2):
    grid = (SEQ // BT,)
    return pl.pallas_call(
        _moe_body,
        grid=grid,
        in_specs=[
            pl.BlockSpec((BT, D_MODEL), lambda t: (t, 0)),
            pl.BlockSpec((BT, NUM_EXPERTS), lambda t: (t, 0)),
            pl.BlockSpec((NUM_EXPERTS, D_FF, D_MODEL), lambda t: (0, 0, 0)),
            pl.BlockSpec((NUM_EXPERTS, D_FF), lambda t: (0, 0)),
            pl.BlockSpec((NUM_EXPERTS, D_MODEL, D_FF), lambda t: (0, 0, 0)),
            pl.BlockSpec((NUM_EXPERTS, D_MODEL), lambda t: (0, 0)),
        ],
        out_specs=pl.BlockSpec((BT, D_MODEL), lambda t: (t, 0)),
        out_shape=jax.ShapeDtypeStruct((SEQ, D_MODEL), jnp.float32),
        compiler_params=pltpu.CompilerParams(
            dimension_semantics=("arbitrary",)),
    )(xf, fw, W1, b1, W2, b2)


@jax.jit
def kernel(x, Wg, W1, b1, W2, b2):
    xh = x.reshape(SEQ, NUM_HEADS, HEAD_DIM).transpose(1, 0, 2)
    attn = _attention(xh).transpose(1, 0, 2).reshape(SEQ, D_MODEL)
    fw = _gating(attn, Wg)
    out = _moe_dense(attn, fw, W1, b1, W2, b2)
    return out.reshape(1, SEQ, D_MODEL)
